# Initial kernel scaffold; baseline (speedup 1.0000x reference)
import jax
import jax.numpy as jnp
from jax.experimental import pallas as pl


def kernel(node_features, edge_index, edge_features, graph_index, embed_W, embed_b, gate_W, gate_b, cand_W, cand_b, bn_g, bn_b, mlp1_W, mlp1_b, mlp2_W, mlp2_b, mlp3_W, mlp3_b):
    raise NotImplementedError("write your pallas kernel here")


# speedup vs baseline: 1.7632x; 1.7632x over previous
import functools

import jax
import jax.numpy as jnp
from jax import lax
from jax.experimental import pallas as pl
from jax.experimental.pallas import tpu as pltpu
from jax.experimental.pallas import tpu_sc as plsc

N = 50000
E = 800000
F = 128
FE = 16
H = 64
B = 64
NC = 3

NWK = 32
EW = E // NWK
CG = 128
NCH = EW // CG
GT = EW - NCH * CG

ET = E // 16
CS = 128
NCS = ET // CS
STL = ET - NCS * CS
HALF = 25088
NPAD = 2 * HALF
AR = 26624
TRASH = HALF
ZR = AR // 16
WR = HALF // 16

_f32 = jnp.float32


@functools.partial(
    pl.kernel,
    out_type=[
        jax.ShapeDtypeStruct((E, 2 * H), _f32),
        jax.ShapeDtypeStruct((E, 2 * H), _f32),
    ],
    mesh=plsc.VectorSubcoreMesh(core_axis_name="c", subcore_axis_name="s"),
    scratch_types=[
        pltpu.VMEM((CG,), jnp.int32),
        pltpu.VMEM((CG,), jnp.int32),
        pltpu.VMEM((CG, 2 * H), _f32),
        pltpu.VMEM((CG, 2 * H), _f32),
        pltpu.VMEM((GT,), jnp.int32),
        pltpu.VMEM((GT,), jnp.int32),
        pltpu.VMEM((GT, 2 * H), _f32),
        pltpu.VMEM((GT, 2 * H), _f32),
        pltpu.SemaphoreType.DMA,
        pltpu.SemaphoreType.DMA,
    ],
)
def _sc_gather(src_h, dst_h, p_h, q_h, s1_h, s2_h, isv, idv, ra, rb,
               isv2, idv2, ra2, rb2, sema, semb):
    c = lax.axis_index("c")
    s = lax.axis_index("s")
    base0 = (s * 2 + c) * EW

    def chunk(base, iv, jv, bufa, bufb, width):
        pltpu.sync_copy(src_h.at[pl.ds(base, width)], iv)
        pltpu.sync_copy(dst_h.at[pl.ds(base, width)], jv)
        cpa = pltpu.async_copy(p_h.at[iv], bufa, sema)
        cpb = pltpu.async_copy(q_h.at[jv], bufb, semb)
        cpa.wait()
        pltpu.sync_copy(bufa, s1_h.at[pl.ds(base, width)])
        cpb.wait()
        pltpu.sync_copy(bufb, s2_h.at[pl.ds(base, width)])

    def body(t, carry):
        chunk(base0 + t * CG, isv, idv, ra, rb, CG)
        return carry

    lax.fori_loop(0, NCH, body, 0)
    chunk(base0 + NCH * CG, isv2, idv2, ra2, rb2, GT)


HH = H // 2


@functools.partial(
    pl.kernel,
    out_type=[
        jax.ShapeDtypeStruct((NPAD, HH), _f32),
        jax.ShapeDtypeStruct((NPAD, HH), _f32),
    ],
    mesh=plsc.VectorSubcoreMesh(core_axis_name="c", subcore_axis_name="s"),
    scratch_types=[
        pltpu.VMEM((CS,), jnp.int32),
        pltpu.VMEM((CS,), jnp.int32),
        pltpu.VMEM((CS, HH), _f32),
        pltpu.VMEM((STL,), jnp.int32),
        pltpu.VMEM((STL,), jnp.int32),
        pltpu.VMEM((STL, HH), _f32),
        pltpu.VMEM((CS, HH), _f32),
        pltpu.VMEM((CS,), jnp.int32),
        pltpu.VMEM((32,), jnp.int32),
        pltpu.VMEM_SHARED((AR, HH), _f32),
    ],
)
def _sc_scatter(src_h, ma_h, mb_h, z_h, agga_h, aggb_h, sbuf, lbuf, mbuf,
                sbuf2, lbuf2, mbuf2, zbuf, ibuf, itail, accum):
    c = lax.axis_index("c")
    s = lax.axis_index("s")
    nbase = c * HALF
    iota16 = lax.iota(jnp.int32, 16)
    pltpu.sync_copy(z_h, zbuf)

    def fill_idx(ref, base, n16):
        for j in range(n16):
            ref[pl.ds(j * 16, 16)] = iota16 + (base + j * 16)

    def chunk(m_h, ebase, sb, lb, mb, width):
        pltpu.sync_copy(src_h.at[pl.ds(ebase, width)], sb)
        pltpu.sync_copy(m_h.at[pl.ds(ebase, width)], mb)
        for j in range(width // 16):
            v = sb[pl.ds(j * 16, 16)]
            li = v - nbase
            ok = (li >= 0) & (li < HALF)
            lb[pl.ds(j * 16, 16)] = jnp.where(ok, li, TRASH)
        pltpu.sync_copy(mb, accum.at[lb], add=True)

    for ph in range(2):
        m_h = ma_h if ph == 0 else mb_h
        agg_h = agga_h if ph == 0 else aggb_h
        for k in range(ZR // CS):
            fill_idx(ibuf, s * ZR + k * CS, CS // 16)
            pltpu.sync_copy(zbuf, accum.at[ibuf])
        plsc.subcore_barrier()

        def body(t, carry):
            chunk(m_h, s * ET + t * CS, sbuf, lbuf, mbuf, CS)
            return carry

        lax.fori_loop(0, NCS, body, 0)
        chunk(m_h, s * ET + NCS * CS, sbuf2, lbuf2, mbuf2, STL)
        plsc.subcore_barrier()
        for k in range(WR // CS):
            fill_idx(ibuf, s * WR + k * CS, CS // 16)
            pltpu.sync_copy(accum.at[ibuf], mbuf)
            pltpu.sync_copy(mbuf, agg_h.at[pl.ds(nbase + s * WR + k * CS, CS)])
        tb = s * WR + (WR // CS) * CS
        fill_idx(itail, tb, 2)
        pltpu.sync_copy(accum.at[itail], mbuf2.at[pl.ds(0, 32)])
        pltpu.sync_copy(mbuf2.at[pl.ds(0, 32)], agg_h.at[pl.ds(nbase + tb, 32)])
        plsc.subcore_barrier()


def _softplus(x):
    return jnp.logaddexp(x, 0.0)


def _embed_body(x_ref, we_ref, be_ref, wp_ref, wq_ref, h_ref, p_ref, q_ref):
    h = jnp.dot(x_ref[...], we_ref[...], preferred_element_type=_f32) + be_ref[...]
    h_ref[...] = h
    p_ref[...] = jnp.dot(h, wp_ref[...], preferred_element_type=_f32)
    q_ref[...] = jnp.dot(h, wq_ref[...], preferred_element_type=_f32)


def _embed_fused(x, we, be, wp, wq):
    bm = 2000
    grid = (N // bm,)
    return pl.pallas_call(
        _embed_body,
        grid=grid,
        in_specs=[
            pl.BlockSpec((bm, F), lambda i: (i, 0)),
            pl.BlockSpec((F, H), lambda i: (0, 0)),
            pl.BlockSpec((1, H), lambda i: (0, 0)),
            pl.BlockSpec((H, 2 * H), lambda i: (0, 0)),
            pl.BlockSpec((H, 2 * H), lambda i: (0, 0)),
        ],
        out_specs=[
            pl.BlockSpec((bm, H), lambda i: (i, 0)),
            pl.BlockSpec((bm, 2 * H), lambda i: (i, 0)),
            pl.BlockSpec((bm, 2 * H), lambda i: (i, 0)),
        ],
        out_shape=[
            jax.ShapeDtypeStruct((N, H), _f32),
            jax.ShapeDtypeStruct((N, 2 * H), _f32),
            jax.ShapeDtypeStruct((N, 2 * H), _f32),
        ],
    )(x, we, be, wp, wq)


def _edge_body(s1_ref, s2_ref, ef_ref, weg_ref, wec_ref, bg_ref, bc_ref, m_ref):
    s1 = s1_ref[...]
    s2 = s2_ref[...]
    ef = ef_ref[...]
    zg = (
        s1[:, :H]
        + s2[:, :H]
        + jnp.dot(ef, weg_ref[...], preferred_element_type=_f32)
        + bg_ref[...]
    )
    zc = (
        s1[:, H:]
        + s2[:, H:]
        + jnp.dot(ef, wec_ref[...], preferred_element_type=_f32)
        + bc_ref[...]
    )
    m = jax.nn.sigmoid(zg) * _softplus(zc)
    m_ref[0][...] = m[:, :HH]
    m_ref[1][...] = m[:, HH:]


def _edge_elementwise(s1, s2, ef, weg, wec, bg, bc):
    bm = 4000
    grid = (E // bm,)
    return pl.pallas_call(
        lambda s1r, s2r, efr, wegr, wecr, bgr, bcr, ma, mb: _edge_body(
            s1r, s2r, efr, wegr, wecr, bgr, bcr, (ma, mb)),
        grid=grid,
        in_specs=[
            pl.BlockSpec((bm, 2 * H), lambda i: (i, 0)),
            pl.BlockSpec((bm, 2 * H), lambda i: (i, 0)),
            pl.BlockSpec((bm, FE), lambda i: (i, 0)),
            pl.BlockSpec((FE, H), lambda i: (0, 0)),
            pl.BlockSpec((FE, H), lambda i: (0, 0)),
            pl.BlockSpec((1, H), lambda i: (0, 0)),
            pl.BlockSpec((1, H), lambda i: (0, 0)),
        ],
        out_specs=[
            pl.BlockSpec((bm, HH), lambda i: (i, 0)),
            pl.BlockSpec((bm, HH), lambda i: (i, 0)),
        ],
        out_shape=[
            jax.ShapeDtypeStruct((E, HH), _f32),
            jax.ShapeDtypeStruct((E, HH), _f32),
        ],
    )(s1, s2, ef, weg, wec, bg, bc)


def _stats_body(agga_ref, aggb_ref, out_ref):
    @pl.when(pl.program_id(0) == 0)
    def _():
        out_ref[...] = jnp.zeros((8, H), _f32)

    a = jnp.concatenate([agga_ref[...], aggb_ref[...]], axis=1)
    s = jnp.sum(a, axis=0, keepdims=True)
    s2 = jnp.sum(a * a, axis=0, keepdims=True)
    out_ref[...] += jnp.concatenate([s, s2, jnp.zeros((6, H), _f32)], axis=0)


def _bn_stats(agga, aggb):
    bm = 2000
    grid = (N // bm,)
    return pl.pallas_call(
        _stats_body,
        grid=grid,
        in_specs=[
            pl.BlockSpec((bm, HH), lambda i: (i, 0)),
            pl.BlockSpec((bm, HH), lambda i: (i, 0)),
        ],
        out_specs=pl.BlockSpec((8, H), lambda i: (0, 0)),
        out_shape=jax.ShapeDtypeStruct((8, H), _f32),
    )(agga, aggb)


def _update_body(h_ref, agga_ref, aggb_ref, st_ref, g_ref, b_ref, wp_ref,
                 wq_ref, hn_ref, p_ref, q_ref):
    st = st_ref[...]
    mu = st[0:1, :] / N
    ex2 = st[1:2, :] / N
    var = ex2 - mu * mu
    inv = lax.rsqrt(var + 1e-5)
    agg = jnp.concatenate([agga_ref[...], aggb_ref[...]], axis=1)
    hn = _softplus(h_ref[...] + (agg - mu) * inv * g_ref[...] + b_ref[...])
    hn_ref[...] = hn
    if p_ref is not None:
        p_ref[...] = jnp.dot(hn, wp_ref[...], preferred_element_type=_f32)
        q_ref[...] = jnp.dot(hn, wq_ref[...], preferred_element_type=_f32)


def _update_fused(h, agga, aggb, st, g, b, wp, wq):
    bm = 2000
    grid = (N // bm,)
    return pl.pallas_call(
        _update_body,
        grid=grid,
        in_specs=[
            pl.BlockSpec((bm, H), lambda i: (i, 0)),
            pl.BlockSpec((bm, HH), lambda i: (i, 0)),
            pl.BlockSpec((bm, HH), lambda i: (i, 0)),
            pl.BlockSpec((8, H), lambda i: (0, 0)),
            pl.BlockSpec((1, H), lambda i: (0, 0)),
            pl.BlockSpec((1, H), lambda i: (0, 0)),
            pl.BlockSpec((H, 2 * H), lambda i: (0, 0)),
            pl.BlockSpec((H, 2 * H), lambda i: (0, 0)),
        ],
        out_specs=[
            pl.BlockSpec((bm, H), lambda i: (i, 0)),
            pl.BlockSpec((bm, 2 * H), lambda i: (i, 0)),
            pl.BlockSpec((bm, 2 * H), lambda i: (i, 0)),
        ],
        out_shape=[
            jax.ShapeDtypeStruct((N, H), _f32),
            jax.ShapeDtypeStruct((N, 2 * H), _f32),
            jax.ShapeDtypeStruct((N, 2 * H), _f32),
        ],
    )(h, agga, aggb, st, g, b, wp, wq)


def _update_last_body(h_ref, agga_ref, aggb_ref, st_ref, g_ref, b_ref, hn_ref):
    _update_body(h_ref, agga_ref, aggb_ref, st_ref, g_ref, b_ref, None, None,
                 hn_ref, None, None)


def _update_last(h, agga, aggb, st, g, b):
    bm = 2000
    grid = (N // bm,)
    return pl.pallas_call(
        _update_last_body,
        grid=grid,
        in_specs=[
            pl.BlockSpec((bm, H), lambda i: (i, 0)),
            pl.BlockSpec((bm, HH), lambda i: (i, 0)),
            pl.BlockSpec((bm, HH), lambda i: (i, 0)),
            pl.BlockSpec((8, H), lambda i: (0, 0)),
            pl.BlockSpec((1, H), lambda i: (0, 0)),
            pl.BlockSpec((1, H), lambda i: (0, 0)),
        ],
        out_specs=pl.BlockSpec((bm, H), lambda i: (i, 0)),
        out_shape=jax.ShapeDtypeStruct((N, H), _f32),
    )(h, agga, aggb, st, g, b)


def _pool_body(h_ref, gi_ref, pool_ref, cnt_ref):
    @pl.when(pl.program_id(0) == 0)
    def _():
        pool_ref[...] = jnp.zeros((B, H), _f32)
        cnt_ref[...] = jnp.zeros((B, H), _f32)

    gi2 = gi_ref[...][0]
    seg = lax.broadcasted_iota(jnp.int32, (B, 1), 0)
    onehot_t = (seg == gi2).astype(_f32)
    pool_ref[...] += jnp.dot(onehot_t, h_ref[...], preferred_element_type=_f32)
    cnt_col = jnp.sum(onehot_t, axis=1, keepdims=True)
    cnt_ref[...] += jnp.broadcast_to(cnt_col, (B, H))


def _pool(h, gi3):
    bm = 2000
    grid = (N // bm,)
    return pl.pallas_call(
        _pool_body,
        grid=grid,
        in_specs=[
            pl.BlockSpec((bm, H), lambda i: (i, 0)),
            pl.BlockSpec((1, 1, bm), lambda i: (i, 0, 0)),
        ],
        out_specs=[
            pl.BlockSpec((B, H), lambda i: (0, 0)),
            pl.BlockSpec((B, H), lambda i: (0, 0)),
        ],
        out_shape=[
            jax.ShapeDtypeStruct((B, H), _f32),
            jax.ShapeDtypeStruct((B, H), _f32),
        ],
    )(h, gi3)


def _mlp_body(pool_ref, cnt_ref, w1_ref, b1_ref, w2_ref, b2_ref, w3_ref, b3_ref,
              out_ref):
    cnt = jnp.maximum(cnt_ref[...], 1.0)[:, 0:1]
    x = pool_ref[...] / cnt
    x1 = _softplus(jnp.dot(x, w1_ref[...], preferred_element_type=_f32) + b1_ref[...])
    x2 = _softplus(jnp.dot(x1, w2_ref[...], preferred_element_type=_f32) + b2_ref[...])
    out_ref[...] = jnp.dot(x2, w3_ref[...], preferred_element_type=_f32) + b3_ref[...]


def _mlp(pool, cnt, w1, b1, w2, b2, w3p, b3p):
    return pl.pallas_call(
        _mlp_body,
        out_shape=jax.ShapeDtypeStruct((B, 128), _f32),
    )(pool, cnt, w1, b1, w2, b2, w3p, b3p)


def kernel(node_features, edge_index, edge_features, graph_index, embed_W,
           embed_b, gate_W, gate_b, cand_W, cand_b, bn_g, bn_b, mlp1_W,
           mlp1_b, mlp2_W, mlp2_b, mlp3_W, mlp3_b):
    src = edge_index[0]
    dst = edge_index[1]

    wps = [jnp.concatenate([gate_W[l, :H, :], cand_W[l, :H, :]], axis=1)
           for l in range(NC)]
    wqs = [jnp.concatenate([gate_W[l, H:2 * H, :], cand_W[l, H:2 * H, :]], axis=1)
           for l in range(NC)]
    wegs = [gate_W[l, 2 * H:, :] for l in range(NC)]
    wecs = [cand_W[l, 2 * H:, :] for l in range(NC)]

    zrows = jnp.zeros((CS, HH), _f32)
    gi3 = graph_index.reshape(N // 2000, 1, 2000)

    h, p, q = _embed_fused(node_features, embed_W, embed_b.reshape(1, H),
                           wps[0], wqs[0])

    for l in range(NC):
        s1, s2 = _sc_gather(src, dst, p, q)
        ma, mb = _edge_elementwise(s1, s2, edge_features, wegs[l], wecs[l],
                                   gate_b[l].reshape(1, H),
                                   cand_b[l].reshape(1, H))
        agga, aggb = _sc_scatter(src, ma, mb, zrows)
        st = _bn_stats(agga, aggb)
        gl = bn_g[l].reshape(1, H)
        bl = bn_b[l].reshape(1, H)
        if l < NC - 1:
            h, p, q = _update_fused(h, agga, aggb, st, gl, bl,
                                    wps[l + 1], wqs[l + 1])
        else:
            h = _update_last(h, agga, aggb, st, gl, bl)

    pool, cnt = _pool(h, gi3)
    w3p = jnp.pad(mlp3_W, ((0, 0), (0, 127)))
    b3p = jnp.pad(mlp3_b, ((0, 127))).reshape(1, 128)
    y = _mlp(pool, cnt, mlp1_W, mlp1_b.reshape(1, 128), mlp2_W,
             mlp2_b.reshape(1, H), w3p, b3p)
    return y[:, 0]


# speedup vs baseline: 2.0209x; 1.1462x over previous
import functools

import jax
import jax.numpy as jnp
from jax import lax
from jax.experimental import pallas as pl
from jax.experimental.pallas import tpu as pltpu
from jax.experimental.pallas import tpu_sc as plsc

N = 50000
E = 800000
F = 128
FE = 16
H = 64
B = 64
NC = 3

NWK = 32
EW = E // NWK
CG = 128
NCH = EW // CG
GT = EW - NCH * CG

ET = E // 16
CS = 128
NCS = ET // CS
STL = ET - NCS * CS
HALF = 25088
NPAD = 2 * HALF
AR = 26624
TRASH = HALF
ZR = AR // 16
WR = HALF // 16

_f32 = jnp.float32


@functools.partial(
    pl.kernel,
    out_type=[
        jax.ShapeDtypeStruct((E, 2 * H), _f32),
        jax.ShapeDtypeStruct((E, 2 * H), _f32),
    ],
    mesh=plsc.VectorSubcoreMesh(core_axis_name="c", subcore_axis_name="s"),
    scratch_types=[
        pltpu.VMEM((CG,), jnp.int32),
        pltpu.VMEM((CG,), jnp.int32),
        pltpu.VMEM((CG, 2 * H), _f32),
        pltpu.VMEM((CG, 2 * H), _f32),
        pltpu.VMEM((GT,), jnp.int32),
        pltpu.VMEM((GT,), jnp.int32),
        pltpu.VMEM((GT, 2 * H), _f32),
        pltpu.VMEM((GT, 2 * H), _f32),
        pltpu.SemaphoreType.DMA,
        pltpu.SemaphoreType.DMA,
    ],
)
def _sc_gather(src_h, dst_h, p_h, q_h, s1_h, s2_h, isv, idv, ra, rb,
               isv2, idv2, ra2, rb2, sema, semb):
    c = lax.axis_index("c")
    s = lax.axis_index("s")
    base0 = (s * 2 + c) * EW

    def chunk(base, iv, jv, bufa, bufb, width):
        pltpu.sync_copy(src_h.at[pl.ds(base, width)], iv)
        pltpu.sync_copy(dst_h.at[pl.ds(base, width)], jv)
        cpa = pltpu.async_copy(p_h.at[iv], bufa, sema)
        cpb = pltpu.async_copy(q_h.at[jv], bufb, semb)
        cpa.wait()
        pltpu.sync_copy(bufa, s1_h.at[pl.ds(base, width)])
        cpb.wait()
        pltpu.sync_copy(bufb, s2_h.at[pl.ds(base, width)])

    def body(t, carry):
        chunk(base0 + t * CG, isv, idv, ra, rb, CG)
        return carry

    lax.fori_loop(0, NCH, body, 0)
    chunk(base0 + NCH * CG, isv2, idv2, ra2, rb2, GT)


HH = H // 2


@functools.partial(
    pl.kernel,
    out_type=jax.ShapeDtypeStruct((2, NPAD, HH), _f32),
    mesh=plsc.VectorSubcoreMesh(core_axis_name="c", subcore_axis_name="s"),
    scratch_types=[
        pltpu.VMEM((CS,), jnp.int32),
        pltpu.VMEM((CS,), jnp.int32),
        pltpu.VMEM((CS, HH), _f32),
        pltpu.VMEM((CS,), jnp.int32),
        pltpu.VMEM((CS,), jnp.int32),
        pltpu.VMEM((CS, HH), _f32),
        pltpu.VMEM((STL,), jnp.int32),
        pltpu.VMEM((STL,), jnp.int32),
        pltpu.VMEM((STL, HH), _f32),
        pltpu.VMEM((CS, HH), _f32),
        pltpu.VMEM((CS,), jnp.int32),
        pltpu.VMEM((32,), jnp.int32),
        pltpu.VMEM_SHARED((AR, HH), _f32),
        pltpu.SemaphoreType.DMA,
        pltpu.SemaphoreType.DMA,
    ],
)
def _sc_scatter(src_h, m3_h, z_h, agg3_h, sbufa, lbufa, mbufa, sbufb, lbufb,
                mbufb, sbuf2, lbuf2, mbuf2, zbuf, ibuf, itail, accum,
                sema, semb):
    c = lax.axis_index("c")
    s = lax.axis_index("s")
    nbase = c * HALF
    iota16 = lax.iota(jnp.int32, 16)
    pltpu.sync_copy(z_h, zbuf)

    def fill_idx(ref, base, n16):
        for j in range(n16):
            ref[pl.ds(j * 16, 16)] = iota16 + (base + j * 16)

    def transform(sb, lb, width):
        for j in range(width // 16):
            v = sb[pl.ds(j * 16, 16)]
            li = v - nbase
            ok = (li >= 0) & (li < HALF)
            lb[pl.ds(j * 16, 16)] = jnp.where(ok, li, TRASH)

    for ph in range(2):
        for k in range(ZR // CS):
            fill_idx(ibuf, s * ZR + k * CS, CS // 16)
            pltpu.sync_copy(zbuf, accum.at[ibuf])
        plsc.subcore_barrier()

        def pair(t2, carry):
            e0 = s * ET + (2 * t2) * CS
            e1 = e0 + CS
            da1 = pltpu.async_copy(src_h.at[pl.ds(e0, CS)], sbufa, sema)
            da2 = pltpu.async_copy(m3_h.at[ph, pl.ds(e0, CS)], mbufa, sema)
            db1 = pltpu.async_copy(src_h.at[pl.ds(e1, CS)], sbufb, semb)
            db2 = pltpu.async_copy(m3_h.at[ph, pl.ds(e1, CS)], mbufb, semb)
            da1.wait()
            da2.wait()
            transform(sbufa, lbufa, CS)
            pltpu.sync_copy(mbufa, accum.at[lbufa], add=True)
            db1.wait()
            db2.wait()
            transform(sbufb, lbufb, CS)
            pltpu.sync_copy(mbufb, accum.at[lbufb], add=True)
            return carry

        lax.fori_loop(0, NCS // 2, pair, 0)
        et = s * ET + NCS * CS
        pltpu.sync_copy(src_h.at[pl.ds(et, STL)], sbuf2)
        pltpu.sync_copy(m3_h.at[ph, pl.ds(et, STL)], mbuf2)
        transform(sbuf2, lbuf2, STL)
        pltpu.sync_copy(mbuf2, accum.at[lbuf2], add=True)
        plsc.subcore_barrier()
        for k in range(WR // CS):
            fill_idx(ibuf, s * WR + k * CS, CS // 16)
            pltpu.sync_copy(accum.at[ibuf], mbufa)
            pltpu.sync_copy(
                mbufa, agg3_h.at[ph, pl.ds(nbase + s * WR + k * CS, CS)])
        tb = s * WR + (WR // CS) * CS
        tw = WR - (WR // CS) * CS
        fill_idx(itail, tb, tw // 16)
        pltpu.sync_copy(accum.at[itail], mbufa.at[pl.ds(0, tw)])
        pltpu.sync_copy(mbufa.at[pl.ds(0, tw)],
                        agg3_h.at[ph, pl.ds(nbase + tb, tw)])
        plsc.subcore_barrier()


def _softplus(x):
    return jnp.logaddexp(x, 0.0)


def _embed_body(x_ref, we_ref, be_ref, wp_ref, wq_ref, h_ref, p_ref, q_ref):
    h = jnp.dot(x_ref[...], we_ref[...], preferred_element_type=_f32) + be_ref[...]
    h_ref[...] = h
    p_ref[...] = jnp.dot(h, wp_ref[...], preferred_element_type=_f32)
    q_ref[...] = jnp.dot(h, wq_ref[...], preferred_element_type=_f32)


def _embed_fused(x, we, be, wp, wq):
    bm = 2000
    grid = (N // bm,)
    return pl.pallas_call(
        _embed_body,
        grid=grid,
        in_specs=[
            pl.BlockSpec((bm, F), lambda i: (i, 0)),
            pl.BlockSpec((F, H), lambda i: (0, 0)),
            pl.BlockSpec((1, H), lambda i: (0, 0)),
            pl.BlockSpec((H, 2 * H), lambda i: (0, 0)),
            pl.BlockSpec((H, 2 * H), lambda i: (0, 0)),
        ],
        out_specs=[
            pl.BlockSpec((bm, H), lambda i: (i, 0)),
            pl.BlockSpec((bm, 2 * H), lambda i: (i, 0)),
            pl.BlockSpec((bm, 2 * H), lambda i: (i, 0)),
        ],
        out_shape=[
            jax.ShapeDtypeStruct((N, H), _f32),
            jax.ShapeDtypeStruct((N, 2 * H), _f32),
            jax.ShapeDtypeStruct((N, 2 * H), _f32),
        ],
    )(x, we, be, wp, wq)


def _edge_body(s1_ref, s2_ref, ef_ref, weg_ref, wec_ref, bg_ref, bc_ref, m_ref):
    s1 = s1_ref[...]
    s2 = s2_ref[...]
    ef = ef_ref[...]
    zg = (
        s1[:, :H]
        + s2[:, :H]
        + jnp.dot(ef, weg_ref[...], preferred_element_type=_f32)
        + bg_ref[...]
    )
    zc = (
        s1[:, H:]
        + s2[:, H:]
        + jnp.dot(ef, wec_ref[...], preferred_element_type=_f32)
        + bc_ref[...]
    )
    m = jax.nn.sigmoid(zg) * _softplus(zc)
    m_ref[...] = jnp.stack([m[:, :HH], m[:, HH:]], axis=0)


def _edge_elementwise(s1, s2, ef, weg, wec, bg, bc):
    bm = 4000
    grid = (E // bm,)
    return pl.pallas_call(
        _edge_body,
        grid=grid,
        in_specs=[
            pl.BlockSpec((bm, 2 * H), lambda i: (i, 0)),
            pl.BlockSpec((bm, 2 * H), lambda i: (i, 0)),
            pl.BlockSpec((bm, FE), lambda i: (i, 0)),
            pl.BlockSpec((FE, H), lambda i: (0, 0)),
            pl.BlockSpec((FE, H), lambda i: (0, 0)),
            pl.BlockSpec((1, H), lambda i: (0, 0)),
            pl.BlockSpec((1, H), lambda i: (0, 0)),
        ],
        out_specs=pl.BlockSpec((2, bm, HH), lambda i: (0, i, 0)),
        out_shape=jax.ShapeDtypeStruct((2, E, HH), _f32),
    )(s1, s2, ef, weg, wec, bg, bc)


def _stats_body(agga_ref, aggb_ref, out_ref):
    @pl.when(pl.program_id(0) == 0)
    def _():
        out_ref[...] = jnp.zeros((8, H), _f32)

    a = jnp.concatenate([agga_ref[...][0], aggb_ref[...][0]], axis=1)
    s = jnp.sum(a, axis=0, keepdims=True)
    s2 = jnp.sum(a * a, axis=0, keepdims=True)
    out_ref[...] += jnp.concatenate([s, s2, jnp.zeros((6, H), _f32)], axis=0)


def _bn_stats(agg3):
    bm = 2000
    grid = (N // bm,)
    return pl.pallas_call(
        _stats_body,
        grid=grid,
        in_specs=[
            pl.BlockSpec((1, bm, HH), lambda i: (0, i, 0)),
            pl.BlockSpec((1, bm, HH), lambda i: (1, i, 0)),
        ],
        out_specs=pl.BlockSpec((8, H), lambda i: (0, 0)),
        out_shape=jax.ShapeDtypeStruct((8, H), _f32),
    )(agg3, agg3)


def _update_body(h_ref, agga_ref, aggb_ref, st_ref, g_ref, b_ref, wp_ref,
                 wq_ref, hn_ref, p_ref, q_ref):
    st = st_ref[...]
    mu = st[0:1, :] / N
    ex2 = st[1:2, :] / N
    var = ex2 - mu * mu
    inv = lax.rsqrt(var + 1e-5)
    agg = jnp.concatenate([agga_ref[...][0], aggb_ref[...][0]], axis=1)
    hn = _softplus(h_ref[...] + (agg - mu) * inv * g_ref[...] + b_ref[...])
    hn_ref[...] = hn
    if p_ref is not None:
        p_ref[...] = jnp.dot(hn, wp_ref[...], preferred_element_type=_f32)
        q_ref[...] = jnp.dot(hn, wq_ref[...], preferred_element_type=_f32)


def _update_fused(h, agg3, st, g, b, wp, wq):
    bm = 2000
    grid = (N // bm,)
    return pl.pallas_call(
        _update_body,
        grid=grid,
        in_specs=[
            pl.BlockSpec((bm, H), lambda i: (i, 0)),
            pl.BlockSpec((1, bm, HH), lambda i: (0, i, 0)),
            pl.BlockSpec((1, bm, HH), lambda i: (1, i, 0)),
            pl.BlockSpec((8, H), lambda i: (0, 0)),
            pl.BlockSpec((1, H), lambda i: (0, 0)),
            pl.BlockSpec((1, H), lambda i: (0, 0)),
            pl.BlockSpec((H, 2 * H), lambda i: (0, 0)),
            pl.BlockSpec((H, 2 * H), lambda i: (0, 0)),
        ],
        out_specs=[
            pl.BlockSpec((bm, H), lambda i: (i, 0)),
            pl.BlockSpec((bm, 2 * H), lambda i: (i, 0)),
            pl.BlockSpec((bm, 2 * H), lambda i: (i, 0)),
        ],
        out_shape=[
            jax.ShapeDtypeStruct((N, H), _f32),
            jax.ShapeDtypeStruct((N, 2 * H), _f32),
            jax.ShapeDtypeStruct((N, 2 * H), _f32),
        ],
    )(h, agg3, agg3, st, g, b, wp, wq)


def _update_last_body(h_ref, agga_ref, aggb_ref, st_ref, g_ref, b_ref, hn_ref):
    _update_body(h_ref, agga_ref, aggb_ref, st_ref, g_ref, b_ref, None, None,
                 hn_ref, None, None)


def _update_last(h, agg3, st, g, b):
    bm = 2000
    grid = (N // bm,)
    return pl.pallas_call(
        _update_last_body,
        grid=grid,
        in_specs=[
            pl.BlockSpec((bm, H), lambda i: (i, 0)),
            pl.BlockSpec((1, bm, HH), lambda i: (0, i, 0)),
            pl.BlockSpec((1, bm, HH), lambda i: (1, i, 0)),
            pl.BlockSpec((8, H), lambda i: (0, 0)),
            pl.BlockSpec((1, H), lambda i: (0, 0)),
            pl.BlockSpec((1, H), lambda i: (0, 0)),
        ],
        out_specs=pl.BlockSpec((bm, H), lambda i: (i, 0)),
        out_shape=jax.ShapeDtypeStruct((N, H), _f32),
    )(h, agg3, agg3, st, g, b)


def _pool_body(h_ref, gi_ref, pool_ref, cnt_ref):
    @pl.when(pl.program_id(0) == 0)
    def _():
        pool_ref[...] = jnp.zeros((B, H), _f32)
        cnt_ref[...] = jnp.zeros((B, H), _f32)

    gi2 = gi_ref[...][0]
    seg = lax.broadcasted_iota(jnp.int32, (B, 1), 0)
    onehot_t = (seg == gi2).astype(_f32)
    pool_ref[...] += jnp.dot(onehot_t, h_ref[...], preferred_element_type=_f32)
    cnt_col = jnp.sum(onehot_t, axis=1, keepdims=True)
    cnt_ref[...] += jnp.broadcast_to(cnt_col, (B, H))


def _pool(h, gi3):
    bm = 2000
    grid = (N // bm,)
    return pl.pallas_call(
        _pool_body,
        grid=grid,
        in_specs=[
            pl.BlockSpec((bm, H), lambda i: (i, 0)),
            pl.BlockSpec((1, 1, bm), lambda i: (i, 0, 0)),
        ],
        out_specs=[
            pl.BlockSpec((B, H), lambda i: (0, 0)),
            pl.BlockSpec((B, H), lambda i: (0, 0)),
        ],
        out_shape=[
            jax.ShapeDtypeStruct((B, H), _f32),
            jax.ShapeDtypeStruct((B, H), _f32),
        ],
    )(h, gi3)


def _mlp_body(pool_ref, cnt_ref, w1_ref, b1_ref, w2_ref, b2_ref, w3_ref, b3_ref,
              out_ref):
    cnt = jnp.maximum(cnt_ref[...], 1.0)[:, 0:1]
    x = pool_ref[...] / cnt
    x1 = _softplus(jnp.dot(x, w1_ref[...], preferred_element_type=_f32) + b1_ref[...])
    x2 = _softplus(jnp.dot(x1, w2_ref[...], preferred_element_type=_f32) + b2_ref[...])
    out_ref[...] = jnp.dot(x2, w3_ref[...], preferred_element_type=_f32) + b3_ref[...]


def _mlp(pool, cnt, w1, b1, w2, b2, w3p, b3p):
    return pl.pallas_call(
        _mlp_body,
        out_shape=jax.ShapeDtypeStruct((B, 128), _f32),
    )(pool, cnt, w1, b1, w2, b2, w3p, b3p)


def kernel(node_features, edge_index, edge_features, graph_index, embed_W,
           embed_b, gate_W, gate_b, cand_W, cand_b, bn_g, bn_b, mlp1_W,
           mlp1_b, mlp2_W, mlp2_b, mlp3_W, mlp3_b):
    src = edge_index[0]
    dst = edge_index[1]

    wps = [jnp.concatenate([gate_W[l, :H, :], cand_W[l, :H, :]], axis=1)
           for l in range(NC)]
    wqs = [jnp.concatenate([gate_W[l, H:2 * H, :], cand_W[l, H:2 * H, :]], axis=1)
           for l in range(NC)]
    wegs = [gate_W[l, 2 * H:, :] for l in range(NC)]
    wecs = [cand_W[l, 2 * H:, :] for l in range(NC)]

    zrows = jnp.zeros((CS, HH), _f32)
    gi3 = graph_index.reshape(N // 2000, 1, 2000)

    h, p, q = _embed_fused(node_features, embed_W, embed_b.reshape(1, H),
                           wps[0], wqs[0])

    for l in range(NC):
        s1, s2 = _sc_gather(src, dst, p, q)
        m3 = _edge_elementwise(s1, s2, edge_features, wegs[l], wecs[l],
                               gate_b[l].reshape(1, H),
                               cand_b[l].reshape(1, H))
        agg3 = _sc_scatter(src, m3, zrows)
        st = _bn_stats(agg3)
        gl = bn_g[l].reshape(1, H)
        bl = bn_b[l].reshape(1, H)
        if l < NC - 1:
            h, p, q = _update_fused(h, agg3, st, gl, bl,
                                    wps[l + 1], wqs[l + 1])
        else:
            h = _update_last(h, agg3, st, gl, bl)

    pool, cnt = _pool(h, gi3)
    w3p = jnp.pad(mlp3_W, ((0, 0), (0, 127)))
    b3p = jnp.pad(mlp3_b, ((0, 127))).reshape(1, 128)
    y = _mlp(pool, cnt, mlp1_W, mlp1_b.reshape(1, 128), mlp2_W,
             mlp2_b.reshape(1, H), w3p, b3p)
    return y[:, 0]


# speedup vs baseline: 2.1653x; 1.0714x over previous
import functools

import jax
import jax.numpy as jnp
from jax import lax
from jax.experimental import pallas as pl
from jax.experimental.pallas import tpu as pltpu
from jax.experimental.pallas import tpu_sc as plsc

N = 50000
E = 800000
F = 128
FE = 16
H = 64
B = 64
NC = 3

NWK = 32
EW = E // NWK
CG = 128
NCH = EW // CG
GT = EW - NCH * CG

ET = E // 16
CS = 128
NCS = ET // CS
STL = ET - NCS * CS
HALF = 25088
NPAD = 2 * HALF
AR = 26624
TRASH = HALF
ZR = AR // 16
WR = HALF // 16

_f32 = jnp.float32


@functools.partial(
    pl.kernel,
    out_type=[
        jax.ShapeDtypeStruct((E, 2 * H), _f32),
        jax.ShapeDtypeStruct((E, 2 * H), _f32),
    ],
    mesh=plsc.VectorSubcoreMesh(core_axis_name="c", subcore_axis_name="s"),
    scratch_types=[
        pltpu.VMEM((CG,), jnp.int32),
        pltpu.VMEM((CG,), jnp.int32),
        pltpu.VMEM((CG, 2 * H), _f32),
        pltpu.VMEM((CG, 2 * H), _f32),
        pltpu.VMEM((CG,), jnp.int32),
        pltpu.VMEM((CG,), jnp.int32),
        pltpu.VMEM((CG, 2 * H), _f32),
        pltpu.VMEM((CG, 2 * H), _f32),
        pltpu.VMEM((GT,), jnp.int32),
        pltpu.VMEM((GT,), jnp.int32),
        pltpu.VMEM((GT, 2 * H), _f32),
        pltpu.VMEM((GT, 2 * H), _f32),
        pltpu.SemaphoreType.DMA,
        pltpu.SemaphoreType.DMA,
        pltpu.SemaphoreType.DMA,
        pltpu.SemaphoreType.DMA,
        pltpu.SemaphoreType.DMA,
        pltpu.SemaphoreType.DMA,
        pltpu.SemaphoreType.DMA,
    ],
)
def _sc_gather(src_h, dst_h, p_h, q_h, s1_h, s2_h,
               isva, idva, raa, rba, isvb, idvb, rab, rbb,
               isv2, idv2, ra2, rb2,
               semia, semib, semga, semgb, semgc, semgd, semw):
    c = lax.axis_index("c")
    s = lax.axis_index("s")
    base0 = (s * 2 + c) * EW

    def pair(t2, carry):
        e0 = base0 + (2 * t2) * CG
        e1 = e0 + CG
        ia1 = pltpu.async_copy(src_h.at[pl.ds(e0, CG)], isva, semia)
        ia2 = pltpu.async_copy(dst_h.at[pl.ds(e0, CG)], idva, semia)
        ib1 = pltpu.async_copy(src_h.at[pl.ds(e1, CG)], isvb, semib)
        ib2 = pltpu.async_copy(dst_h.at[pl.ds(e1, CG)], idvb, semib)
        ia1.wait()
        ia2.wait()
        ga = pltpu.async_copy(p_h.at[isva], raa, semga)
        gb = pltpu.async_copy(q_h.at[idva], rba, semgb)
        ib1.wait()
        ib2.wait()
        gc = pltpu.async_copy(p_h.at[isvb], rab, semgc)
        gd = pltpu.async_copy(q_h.at[idvb], rbb, semgd)
        ga.wait()
        w1 = pltpu.async_copy(raa, s1_h.at[pl.ds(e0, CG)], semw)
        gb.wait()
        w2 = pltpu.async_copy(rba, s2_h.at[pl.ds(e0, CG)], semw)
        gc.wait()
        w3 = pltpu.async_copy(rab, s1_h.at[pl.ds(e1, CG)], semw)
        gd.wait()
        w4 = pltpu.async_copy(rbb, s2_h.at[pl.ds(e1, CG)], semw)
        w1.wait()
        w2.wait()
        w3.wait()
        w4.wait()
        return carry

    lax.fori_loop(0, NCH // 2, pair, 0)

    def chunk(base, iv, jv, bufa, bufb, width):
        pltpu.sync_copy(src_h.at[pl.ds(base, width)], iv)
        pltpu.sync_copy(dst_h.at[pl.ds(base, width)], jv)
        cpa = pltpu.async_copy(p_h.at[iv], bufa, semga)
        cpb = pltpu.async_copy(q_h.at[jv], bufb, semgb)
        cpa.wait()
        pltpu.sync_copy(bufa, s1_h.at[pl.ds(base, width)])
        cpb.wait()
        pltpu.sync_copy(bufb, s2_h.at[pl.ds(base, width)])

    if NCH % 2:
        chunk(base0 + (NCH - 1) * CG, isva, idva, raa, rba, CG)
    chunk(base0 + NCH * CG, isv2, idv2, ra2, rb2, GT)


HH = H // 2


@functools.partial(
    pl.kernel,
    out_type=jax.ShapeDtypeStruct((2, NPAD, HH), _f32),
    mesh=plsc.VectorSubcoreMesh(core_axis_name="c", subcore_axis_name="s"),
    scratch_types=[
        pltpu.VMEM((CS,), jnp.int32),
        pltpu.VMEM((CS,), jnp.int32),
        pltpu.VMEM((CS, HH), _f32),
        pltpu.VMEM((CS,), jnp.int32),
        pltpu.VMEM((CS,), jnp.int32),
        pltpu.VMEM((CS, HH), _f32),
        pltpu.VMEM((STL,), jnp.int32),
        pltpu.VMEM((STL,), jnp.int32),
        pltpu.VMEM((STL, HH), _f32),
        pltpu.VMEM((CS, HH), _f32),
        pltpu.VMEM((CS,), jnp.int32),
        pltpu.VMEM((32,), jnp.int32),
        pltpu.VMEM_SHARED((AR, HH), _f32),
        pltpu.SemaphoreType.DMA,
        pltpu.SemaphoreType.DMA,
    ],
)
def _sc_scatter(src_h, m3_h, z_h, agg3_h, sbufa, lbufa, mbufa, sbufb, lbufb,
                mbufb, sbuf2, lbuf2, mbuf2, zbuf, ibuf, itail, accum,
                sema, semb):
    c = lax.axis_index("c")
    s = lax.axis_index("s")
    nbase = c * HALF
    iota16 = lax.iota(jnp.int32, 16)
    pltpu.sync_copy(z_h, zbuf)

    def fill_idx(ref, base, n16):
        for j in range(n16):
            ref[pl.ds(j * 16, 16)] = iota16 + (base + j * 16)

    def transform(sb, lb, width):
        for j in range(width // 16):
            v = sb[pl.ds(j * 16, 16)]
            li = v - nbase
            ok = (li >= 0) & (li < HALF)
            lb[pl.ds(j * 16, 16)] = jnp.where(ok, li, TRASH)

    for ph in range(2):
        for k in range(ZR // CS):
            fill_idx(ibuf, s * ZR + k * CS, CS // 16)
            pltpu.sync_copy(zbuf, accum.at[ibuf])
        plsc.subcore_barrier()

        def pair(t2, carry):
            e0 = s * ET + (2 * t2) * CS
            e1 = e0 + CS
            da1 = pltpu.async_copy(src_h.at[pl.ds(e0, CS)], sbufa, sema)
            da2 = pltpu.async_copy(m3_h.at[ph, pl.ds(e0, CS)], mbufa, sema)
            db1 = pltpu.async_copy(src_h.at[pl.ds(e1, CS)], sbufb, semb)
            db2 = pltpu.async_copy(m3_h.at[ph, pl.ds(e1, CS)], mbufb, semb)
            da1.wait()
            da2.wait()
            transform(sbufa, lbufa, CS)
            pltpu.sync_copy(mbufa, accum.at[lbufa], add=True)
            db1.wait()
            db2.wait()
            transform(sbufb, lbufb, CS)
            pltpu.sync_copy(mbufb, accum.at[lbufb], add=True)
            return carry

        lax.fori_loop(0, NCS // 2, pair, 0)
        et = s * ET + NCS * CS
        pltpu.sync_copy(src_h.at[pl.ds(et, STL)], sbuf2)
        pltpu.sync_copy(m3_h.at[ph, pl.ds(et, STL)], mbuf2)
        transform(sbuf2, lbuf2, STL)
        pltpu.sync_copy(mbuf2, accum.at[lbuf2], add=True)
        plsc.subcore_barrier()
        for k in range(WR // CS):
            fill_idx(ibuf, s * WR + k * CS, CS // 16)
            pltpu.sync_copy(accum.at[ibuf], mbufa)
            pltpu.sync_copy(
                mbufa, agg3_h.at[ph, pl.ds(nbase + s * WR + k * CS, CS)])
        tb = s * WR + (WR // CS) * CS
        tw = WR - (WR // CS) * CS
        fill_idx(itail, tb, tw // 16)
        pltpu.sync_copy(accum.at[itail], mbufa.at[pl.ds(0, tw)])
        pltpu.sync_copy(mbufa.at[pl.ds(0, tw)],
                        agg3_h.at[ph, pl.ds(nbase + tb, tw)])
        plsc.subcore_barrier()


def _softplus(x):
    return jnp.logaddexp(x, 0.0)


def _embed_body(x_ref, we_ref, be_ref, wp_ref, wq_ref, h_ref, p_ref, q_ref):
    h = jnp.dot(x_ref[...], we_ref[...], preferred_element_type=_f32) + be_ref[...]
    h_ref[...] = h
    p_ref[...] = jnp.dot(h, wp_ref[...], preferred_element_type=_f32)
    q_ref[...] = jnp.dot(h, wq_ref[...], preferred_element_type=_f32)


def _embed_fused(x, we, be, wp, wq):
    bm = 2000
    grid = (N // bm,)
    return pl.pallas_call(
        _embed_body,
        grid=grid,
        in_specs=[
            pl.BlockSpec((bm, F), lambda i: (i, 0)),
            pl.BlockSpec((F, H), lambda i: (0, 0)),
            pl.BlockSpec((1, H), lambda i: (0, 0)),
            pl.BlockSpec((H, 2 * H), lambda i: (0, 0)),
            pl.BlockSpec((H, 2 * H), lambda i: (0, 0)),
        ],
        out_specs=[
            pl.BlockSpec((bm, H), lambda i: (i, 0)),
            pl.BlockSpec((bm, 2 * H), lambda i: (i, 0)),
            pl.BlockSpec((bm, 2 * H), lambda i: (i, 0)),
        ],
        out_shape=[
            jax.ShapeDtypeStruct((N, H), _f32),
            jax.ShapeDtypeStruct((N, 2 * H), _f32),
            jax.ShapeDtypeStruct((N, 2 * H), _f32),
        ],
    )(x, we, be, wp, wq)


def _edge_body(s1_ref, s2_ref, ef_ref, weg_ref, wec_ref, bg_ref, bc_ref, m_ref):
    s1 = s1_ref[...]
    s2 = s2_ref[...]
    ef = ef_ref[...]
    zg = (
        s1[:, :H]
        + s2[:, :H]
        + jnp.dot(ef, weg_ref[...], preferred_element_type=_f32)
        + bg_ref[...]
    )
    zc = (
        s1[:, H:]
        + s2[:, H:]
        + jnp.dot(ef, wec_ref[...], preferred_element_type=_f32)
        + bc_ref[...]
    )
    m = jax.nn.sigmoid(zg) * _softplus(zc)
    m_ref[...] = jnp.stack([m[:, :HH], m[:, HH:]], axis=0)


def _edge_elementwise(s1, s2, ef, weg, wec, bg, bc):
    bm = 4000
    grid = (E // bm,)
    return pl.pallas_call(
        _edge_body,
        grid=grid,
        in_specs=[
            pl.BlockSpec((bm, 2 * H), lambda i: (i, 0)),
            pl.BlockSpec((bm, 2 * H), lambda i: (i, 0)),
            pl.BlockSpec((bm, FE), lambda i: (i, 0)),
            pl.BlockSpec((FE, H), lambda i: (0, 0)),
            pl.BlockSpec((FE, H), lambda i: (0, 0)),
            pl.BlockSpec((1, H), lambda i: (0, 0)),
            pl.BlockSpec((1, H), lambda i: (0, 0)),
        ],
        out_specs=pl.BlockSpec((2, bm, HH), lambda i: (0, i, 0)),
        out_shape=jax.ShapeDtypeStruct((2, E, HH), _f32),
    )(s1, s2, ef, weg, wec, bg, bc)


def _stats_body(agga_ref, aggb_ref, out_ref):
    @pl.when(pl.program_id(0) == 0)
    def _():
        out_ref[...] = jnp.zeros((8, H), _f32)

    a = jnp.concatenate([agga_ref[...][0], aggb_ref[...][0]], axis=1)
    s = jnp.sum(a, axis=0, keepdims=True)
    s2 = jnp.sum(a * a, axis=0, keepdims=True)
    out_ref[...] += jnp.concatenate([s, s2, jnp.zeros((6, H), _f32)], axis=0)


def _bn_stats(agg3):
    bm = 2000
    grid = (N // bm,)
    return pl.pallas_call(
        _stats_body,
        grid=grid,
        in_specs=[
            pl.BlockSpec((1, bm, HH), lambda i: (0, i, 0)),
            pl.BlockSpec((1, bm, HH), lambda i: (1, i, 0)),
        ],
        out_specs=pl.BlockSpec((8, H), lambda i: (0, 0)),
        out_shape=jax.ShapeDtypeStruct((8, H), _f32),
    )(agg3, agg3)


def _update_body(h_ref, agga_ref, aggb_ref, st_ref, g_ref, b_ref, wp_ref,
                 wq_ref, hn_ref, p_ref, q_ref):
    st = st_ref[...]
    mu = st[0:1, :] / N
    ex2 = st[1:2, :] / N
    var = ex2 - mu * mu
    inv = lax.rsqrt(var + 1e-5)
    agg = jnp.concatenate([agga_ref[...][0], aggb_ref[...][0]], axis=1)
    hn = _softplus(h_ref[...] + (agg - mu) * inv * g_ref[...] + b_ref[...])
    hn_ref[...] = hn
    if p_ref is not None:
        p_ref[...] = jnp.dot(hn, wp_ref[...], preferred_element_type=_f32)
        q_ref[...] = jnp.dot(hn, wq_ref[...], preferred_element_type=_f32)


def _update_fused(h, agg3, st, g, b, wp, wq):
    bm = 2000
    grid = (N // bm,)
    return pl.pallas_call(
        _update_body,
        grid=grid,
        in_specs=[
            pl.BlockSpec((bm, H), lambda i: (i, 0)),
            pl.BlockSpec((1, bm, HH), lambda i: (0, i, 0)),
            pl.BlockSpec((1, bm, HH), lambda i: (1, i, 0)),
            pl.BlockSpec((8, H), lambda i: (0, 0)),
            pl.BlockSpec((1, H), lambda i: (0, 0)),
            pl.BlockSpec((1, H), lambda i: (0, 0)),
            pl.BlockSpec((H, 2 * H), lambda i: (0, 0)),
            pl.BlockSpec((H, 2 * H), lambda i: (0, 0)),
        ],
        out_specs=[
            pl.BlockSpec((bm, H), lambda i: (i, 0)),
            pl.BlockSpec((bm, 2 * H), lambda i: (i, 0)),
            pl.BlockSpec((bm, 2 * H), lambda i: (i, 0)),
        ],
        out_shape=[
            jax.ShapeDtypeStruct((N, H), _f32),
            jax.ShapeDtypeStruct((N, 2 * H), _f32),
            jax.ShapeDtypeStruct((N, 2 * H), _f32),
        ],
    )(h, agg3, agg3, st, g, b, wp, wq)


def _update_last_body(h_ref, agga_ref, aggb_ref, st_ref, g_ref, b_ref, hn_ref):
    _update_body(h_ref, agga_ref, aggb_ref, st_ref, g_ref, b_ref, None, None,
                 hn_ref, None, None)


def _update_last(h, agg3, st, g, b):
    bm = 2000
    grid = (N // bm,)
    return pl.pallas_call(
        _update_last_body,
        grid=grid,
        in_specs=[
            pl.BlockSpec((bm, H), lambda i: (i, 0)),
            pl.BlockSpec((1, bm, HH), lambda i: (0, i, 0)),
            pl.BlockSpec((1, bm, HH), lambda i: (1, i, 0)),
            pl.BlockSpec((8, H), lambda i: (0, 0)),
            pl.BlockSpec((1, H), lambda i: (0, 0)),
            pl.BlockSpec((1, H), lambda i: (0, 0)),
        ],
        out_specs=pl.BlockSpec((bm, H), lambda i: (i, 0)),
        out_shape=jax.ShapeDtypeStruct((N, H), _f32),
    )(h, agg3, agg3, st, g, b)


def _pool_body(h_ref, gi_ref, pool_ref, cnt_ref):
    @pl.when(pl.program_id(0) == 0)
    def _():
        pool_ref[...] = jnp.zeros((B, H), _f32)
        cnt_ref[...] = jnp.zeros((B, H), _f32)

    gi2 = gi_ref[...][0]
    seg = lax.broadcasted_iota(jnp.int32, (B, 1), 0)
    onehot_t = (seg == gi2).astype(_f32)
    pool_ref[...] += jnp.dot(onehot_t, h_ref[...], preferred_element_type=_f32)
    cnt_col = jnp.sum(onehot_t, axis=1, keepdims=True)
    cnt_ref[...] += jnp.broadcast_to(cnt_col, (B, H))


def _pool(h, gi3):
    bm = 2000
    grid = (N // bm,)
    return pl.pallas_call(
        _pool_body,
        grid=grid,
        in_specs=[
            pl.BlockSpec((bm, H), lambda i: (i, 0)),
            pl.BlockSpec((1, 1, bm), lambda i: (i, 0, 0)),
        ],
        out_specs=[
            pl.BlockSpec((B, H), lambda i: (0, 0)),
            pl.BlockSpec((B, H), lambda i: (0, 0)),
        ],
        out_shape=[
            jax.ShapeDtypeStruct((B, H), _f32),
            jax.ShapeDtypeStruct((B, H), _f32),
        ],
    )(h, gi3)


def _mlp_body(pool_ref, cnt_ref, w1_ref, b1_ref, w2_ref, b2_ref, w3_ref, b3_ref,
              out_ref):
    cnt = jnp.maximum(cnt_ref[...], 1.0)[:, 0:1]
    x = pool_ref[...] / cnt
    x1 = _softplus(jnp.dot(x, w1_ref[...], preferred_element_type=_f32) + b1_ref[...])
    x2 = _softplus(jnp.dot(x1, w2_ref[...], preferred_element_type=_f32) + b2_ref[...])
    out_ref[...] = jnp.dot(x2, w3_ref[...], preferred_element_type=_f32) + b3_ref[...]


def _mlp(pool, cnt, w1, b1, w2, b2, w3p, b3p):
    return pl.pallas_call(
        _mlp_body,
        out_shape=jax.ShapeDtypeStruct((B, 128), _f32),
    )(pool, cnt, w1, b1, w2, b2, w3p, b3p)


def kernel(node_features, edge_index, edge_features, graph_index, embed_W,
           embed_b, gate_W, gate_b, cand_W, cand_b, bn_g, bn_b, mlp1_W,
           mlp1_b, mlp2_W, mlp2_b, mlp3_W, mlp3_b):
    src = edge_index[0]
    dst = edge_index[1]

    wps = [jnp.concatenate([gate_W[l, :H, :], cand_W[l, :H, :]], axis=1)
           for l in range(NC)]
    wqs = [jnp.concatenate([gate_W[l, H:2 * H, :], cand_W[l, H:2 * H, :]], axis=1)
           for l in range(NC)]
    wegs = [gate_W[l, 2 * H:, :] for l in range(NC)]
    wecs = [cand_W[l, 2 * H:, :] for l in range(NC)]

    zrows = jnp.zeros((CS, HH), _f32)
    gi3 = graph_index.reshape(N // 2000, 1, 2000)

    h, p, q = _embed_fused(node_features, embed_W, embed_b.reshape(1, H),
                           wps[0], wqs[0])

    for l in range(NC):
        s1, s2 = _sc_gather(src, dst, p, q)
        m3 = _edge_elementwise(s1, s2, edge_features, wegs[l], wecs[l],
                               gate_b[l].reshape(1, H),
                               cand_b[l].reshape(1, H))
        agg3 = _sc_scatter(src, m3, zrows)
        st = _bn_stats(agg3)
        gl = bn_g[l].reshape(1, H)
        bl = bn_b[l].reshape(1, H)
        if l < NC - 1:
            h, p, q = _update_fused(h, agg3, st, gl, bl,
                                    wps[l + 1], wqs[l + 1])
        else:
            h = _update_last(h, agg3, st, gl, bl)

    pool, cnt = _pool(h, gi3)
    w3p = jnp.pad(mlp3_W, ((0, 0), (0, 127)))
    b3p = jnp.pad(mlp3_b, ((0, 127))).reshape(1, 128)
    y = _mlp(pool, cnt, mlp1_W, mlp1_b.reshape(1, 128), mlp2_W,
             mlp2_b.reshape(1, H), w3p, b3p)
    return y[:, 0]


# speedup vs baseline: 2.1706x; 1.0024x over previous
import functools

import jax
import jax.numpy as jnp
from jax import lax
from jax.experimental import pallas as pl
from jax.experimental.pallas import tpu as pltpu
from jax.experimental.pallas import tpu_sc as plsc

N = 50000
E = 800000
F = 128
FE = 16
H = 64
B = 64
NC = 3

NWK = 32
EW = E // NWK
CG = 128
NCH = EW // CG
GT = EW - NCH * CG

ET = E // 16
CS = 128
NCS = ET // CS
STL = ET - NCS * CS
HALF = 25088
NPAD = 2 * HALF
AR = 26624
TRASH = HALF
ZR = AR // 16
WR = HALF // 16

_f32 = jnp.float32


@functools.partial(
    pl.kernel,
    out_type=[
        jax.ShapeDtypeStruct((E, 2 * H), _f32),
        jax.ShapeDtypeStruct((E, 2 * H), _f32),
    ],
    mesh=plsc.VectorSubcoreMesh(core_axis_name="c", subcore_axis_name="s"),
    scratch_types=[
        pltpu.VMEM((CG,), jnp.int32),
        pltpu.VMEM((CG,), jnp.int32),
        pltpu.VMEM((CG, 2 * H), _f32),
        pltpu.VMEM((CG, 2 * H), _f32),
        pltpu.VMEM((CG,), jnp.int32),
        pltpu.VMEM((CG,), jnp.int32),
        pltpu.VMEM((CG, 2 * H), _f32),
        pltpu.VMEM((CG, 2 * H), _f32),
        pltpu.VMEM((GT,), jnp.int32),
        pltpu.VMEM((GT,), jnp.int32),
        pltpu.VMEM((GT, 2 * H), _f32),
        pltpu.VMEM((GT, 2 * H), _f32),
        pltpu.SemaphoreType.DMA,
        pltpu.SemaphoreType.DMA,
        pltpu.SemaphoreType.DMA,
        pltpu.SemaphoreType.DMA,
        pltpu.SemaphoreType.DMA,
        pltpu.SemaphoreType.DMA,
        pltpu.SemaphoreType.DMA,
    ],
)
def _sc_gather(src_h, dst_h, p_h, q_h, s1_h, s2_h,
               isva, idva, raa, rba, isvb, idvb, rab, rbb,
               isv2, idv2, ra2, rb2,
               semia, semib, semga, semgb, semgc, semgd, semw):
    c = lax.axis_index("c")
    s = lax.axis_index("s")
    base0 = (s * 2 + c) * EW

    def pair(t2, carry):
        e0 = base0 + (2 * t2) * CG
        e1 = e0 + CG
        ia1 = pltpu.async_copy(src_h.at[pl.ds(e0, CG)], isva, semia)
        ia2 = pltpu.async_copy(dst_h.at[pl.ds(e0, CG)], idva, semia)
        ib1 = pltpu.async_copy(src_h.at[pl.ds(e1, CG)], isvb, semib)
        ib2 = pltpu.async_copy(dst_h.at[pl.ds(e1, CG)], idvb, semib)
        ia1.wait()
        ia2.wait()
        ga = pltpu.async_copy(p_h.at[isva], raa, semga)
        gb = pltpu.async_copy(q_h.at[idva], rba, semgb)
        ib1.wait()
        ib2.wait()
        gc = pltpu.async_copy(p_h.at[isvb], rab, semgc)
        gd = pltpu.async_copy(q_h.at[idvb], rbb, semgd)
        ga.wait()
        w1 = pltpu.async_copy(raa, s1_h.at[pl.ds(e0, CG)], semw)
        gb.wait()
        w2 = pltpu.async_copy(rba, s2_h.at[pl.ds(e0, CG)], semw)
        gc.wait()
        w3 = pltpu.async_copy(rab, s1_h.at[pl.ds(e1, CG)], semw)
        gd.wait()
        w4 = pltpu.async_copy(rbb, s2_h.at[pl.ds(e1, CG)], semw)
        w1.wait()
        w2.wait()
        w3.wait()
        w4.wait()
        return carry

    lax.fori_loop(0, NCH // 2, pair, 0)

    def chunk(base, iv, jv, bufa, bufb, width):
        pltpu.sync_copy(src_h.at[pl.ds(base, width)], iv)
        pltpu.sync_copy(dst_h.at[pl.ds(base, width)], jv)
        cpa = pltpu.async_copy(p_h.at[iv], bufa, semga)
        cpb = pltpu.async_copy(q_h.at[jv], bufb, semgb)
        cpa.wait()
        pltpu.sync_copy(bufa, s1_h.at[pl.ds(base, width)])
        cpb.wait()
        pltpu.sync_copy(bufb, s2_h.at[pl.ds(base, width)])

    if NCH % 2:
        chunk(base0 + (NCH - 1) * CG, isva, idva, raa, rba, CG)
    chunk(base0 + NCH * CG, isv2, idv2, ra2, rb2, GT)


HH = H // 2


@functools.partial(
    pl.kernel,
    out_type=jax.ShapeDtypeStruct((2, NPAD, HH), _f32),
    mesh=plsc.VectorSubcoreMesh(core_axis_name="c", subcore_axis_name="s"),
    scratch_types=[
        pltpu.VMEM((CS,), jnp.int32),
        pltpu.VMEM((CS, HH), _f32),
        pltpu.VMEM((CS,), jnp.int32),
        pltpu.VMEM((CS, HH), _f32),
        pltpu.VMEM((STL,), jnp.int32),
        pltpu.VMEM((STL, HH), _f32),
        pltpu.VMEM((CS, HH), _f32),
        pltpu.VMEM((CS,), jnp.int32),
        pltpu.VMEM((32,), jnp.int32),
        pltpu.VMEM_SHARED((AR, HH), _f32),
        pltpu.SemaphoreType.DMA,
        pltpu.SemaphoreType.DMA,
        pltpu.SemaphoreType.DMA,
        pltpu.SemaphoreType.DMA,
    ],
)
def _sc_scatter(lidx_h, m3_h, z_h, agg3_h, lb0, mb0, lb1, mb1,
                lbt, mbt, zbuf, ibuf, itail, accum,
                si0, si1, sa0, sa1):
    c = lax.axis_index("c")
    s = lax.axis_index("s")
    iota16 = lax.iota(jnp.int32, 16)
    lbs = (lb0, lb1)
    mbs = (mb0, mb1)
    sis = (si0, si1)
    sas = (sa0, sa1)
    pltpu.sync_copy(z_h, zbuf)

    def fill_idx(ref, base, n16):
        for j in range(n16):
            ref[pl.ds(j * 16, 16)] = iota16 + (base + j * 16)

    for ph in range(2):
        for k in range(ZR // CS):
            fill_idx(ibuf, s * ZR + k * CS, CS // 16)
            pltpu.sync_copy(zbuf, accum.at[ibuf])
        plsc.subcore_barrier()

        def quad(t4, carry):
            e = s * ET + (2 * t4) * CS
            ins = []
            for k in range(2):
                ek = e + k * CS
                ins.append((
                    pltpu.async_copy(lidx_h.at[pl.ds(c * E + ek, CS)], lbs[k],
                                     sis[k]),
                    pltpu.async_copy(m3_h.at[ph, pl.ds(ek, CS)], mbs[k],
                                     sis[k]),
                ))
            adds = []
            for k in range(2):
                ins[k][0].wait()
                ins[k][1].wait()
                adds.append(pltpu.async_copy(mbs[k], accum.at[lbs[k]],
                                             sas[k], add=True))
            for k in range(2):
                adds[k].wait()
            return carry

        lax.fori_loop(0, NCS // 2, quad, 0)
        for r in range((NCS // 2) * 2, NCS):
            er = s * ET + r * CS
            pltpu.sync_copy(lidx_h.at[pl.ds(c * E + er, CS)], lb0)
            pltpu.sync_copy(m3_h.at[ph, pl.ds(er, CS)], mb0)
            pltpu.sync_copy(mb0, accum.at[lb0], add=True)
        et = s * ET + NCS * CS
        pltpu.sync_copy(lidx_h.at[pl.ds(c * E + et, STL)], lbt)
        pltpu.sync_copy(m3_h.at[ph, pl.ds(et, STL)], mbt)
        pltpu.sync_copy(mbt, accum.at[lbt], add=True)
        plsc.subcore_barrier()
        nbase = c * HALF
        for k in range(WR // CS):
            fill_idx(ibuf, s * WR + k * CS, CS // 16)
            pltpu.sync_copy(accum.at[ibuf], mb0)
            pltpu.sync_copy(
                mb0, agg3_h.at[ph, pl.ds(nbase + s * WR + k * CS, CS)])
        tb = s * WR + (WR // CS) * CS
        tw = WR - (WR // CS) * CS
        fill_idx(itail, tb, tw // 16)
        pltpu.sync_copy(accum.at[itail], mb0.at[pl.ds(0, tw)])
        pltpu.sync_copy(mb0.at[pl.ds(0, tw)],
                        agg3_h.at[ph, pl.ds(nbase + tb, tw)])
        plsc.subcore_barrier()


def _softplus(x):
    return jnp.logaddexp(x, 0.0)


def _lidx_body(src_ref, out_ref):
    sv = src_ref[...][0]
    li0 = jnp.where(sv < HALF, sv, TRASH)
    li1 = jnp.where(sv >= HALF, sv - HALF, TRASH)
    out_ref[...] = jnp.stack([li0, li1], axis=0)


def _lidx(src3):
    bm = 6400
    grid = (E // bm,)
    return pl.pallas_call(
        _lidx_body,
        grid=grid,
        in_specs=[pl.BlockSpec((1, 1, bm), lambda i: (i, 0, 0))],
        out_specs=pl.BlockSpec((2, 1, bm), lambda i: (0, 0, i)),
        out_shape=jax.ShapeDtypeStruct((2, 1, E), jnp.int32),
    )(src3)


def _embed_body(x_ref, we_ref, be_ref, wp_ref, wq_ref, h_ref, p_ref, q_ref):
    h = jnp.dot(x_ref[...], we_ref[...], preferred_element_type=_f32) + be_ref[...]
    h_ref[...] = h
    p_ref[...] = jnp.dot(h, wp_ref[...], preferred_element_type=_f32)
    q_ref[...] = jnp.dot(h, wq_ref[...], preferred_element_type=_f32)


def _embed_fused(x, we, be, wp, wq):
    bm = 2000
    grid = (N // bm,)
    return pl.pallas_call(
        _embed_body,
        grid=grid,
        in_specs=[
            pl.BlockSpec((bm, F), lambda i: (i, 0)),
            pl.BlockSpec((F, H), lambda i: (0, 0)),
            pl.BlockSpec((1, H), lambda i: (0, 0)),
            pl.BlockSpec((H, 2 * H), lambda i: (0, 0)),
            pl.BlockSpec((H, 2 * H), lambda i: (0, 0)),
        ],
        out_specs=[
            pl.BlockSpec((bm, H), lambda i: (i, 0)),
            pl.BlockSpec((bm, 2 * H), lambda i: (i, 0)),
            pl.BlockSpec((bm, 2 * H), lambda i: (i, 0)),
        ],
        out_shape=[
            jax.ShapeDtypeStruct((N, H), _f32),
            jax.ShapeDtypeStruct((N, 2 * H), _f32),
            jax.ShapeDtypeStruct((N, 2 * H), _f32),
        ],
    )(x, we, be, wp, wq)


def _edge_body(s1_ref, s2_ref, ef_ref, weg_ref, wec_ref, bg_ref, bc_ref, m_ref):
    s1 = s1_ref[...]
    s2 = s2_ref[...]
    ef = ef_ref[...]
    zg = (
        s1[:, :H]
        + s2[:, :H]
        + jnp.dot(ef, weg_ref[...], preferred_element_type=_f32)
        + bg_ref[...]
    )
    zc = (
        s1[:, H:]
        + s2[:, H:]
        + jnp.dot(ef, wec_ref[...], preferred_element_type=_f32)
        + bc_ref[...]
    )
    m = jax.nn.sigmoid(zg) * _softplus(zc)
    m_ref[...] = jnp.stack([m[:, :HH], m[:, HH:]], axis=0)


def _edge_elementwise(s1, s2, ef, weg, wec, bg, bc):
    bm = 4000
    grid = (E // bm,)
    return pl.pallas_call(
        _edge_body,
        grid=grid,
        in_specs=[
            pl.BlockSpec((bm, 2 * H), lambda i: (i, 0)),
            pl.BlockSpec((bm, 2 * H), lambda i: (i, 0)),
            pl.BlockSpec((bm, FE), lambda i: (i, 0)),
            pl.BlockSpec((FE, H), lambda i: (0, 0)),
            pl.BlockSpec((FE, H), lambda i: (0, 0)),
            pl.BlockSpec((1, H), lambda i: (0, 0)),
            pl.BlockSpec((1, H), lambda i: (0, 0)),
        ],
        out_specs=pl.BlockSpec((2, bm, HH), lambda i: (0, i, 0)),
        out_shape=jax.ShapeDtypeStruct((2, E, HH), _f32),
    )(s1, s2, ef, weg, wec, bg, bc)


def _stats_body(agga_ref, aggb_ref, out_ref):
    @pl.when(pl.program_id(0) == 0)
    def _():
        out_ref[...] = jnp.zeros((8, H), _f32)

    a = jnp.concatenate([agga_ref[...][0], aggb_ref[...][0]], axis=1)
    s = jnp.sum(a, axis=0, keepdims=True)
    s2 = jnp.sum(a * a, axis=0, keepdims=True)
    out_ref[...] += jnp.concatenate([s, s2, jnp.zeros((6, H), _f32)], axis=0)


def _bn_stats(agg3):
    bm = 2000
    grid = (N // bm,)
    return pl.pallas_call(
        _stats_body,
        grid=grid,
        in_specs=[
            pl.BlockSpec((1, bm, HH), lambda i: (0, i, 0)),
            pl.BlockSpec((1, bm, HH), lambda i: (1, i, 0)),
        ],
        out_specs=pl.BlockSpec((8, H), lambda i: (0, 0)),
        out_shape=jax.ShapeDtypeStruct((8, H), _f32),
    )(agg3, agg3)


def _update_body(h_ref, agga_ref, aggb_ref, st_ref, g_ref, b_ref, wp_ref,
                 wq_ref, hn_ref, p_ref, q_ref):
    st = st_ref[...]
    mu = st[0:1, :] / N
    ex2 = st[1:2, :] / N
    var = ex2 - mu * mu
    inv = lax.rsqrt(var + 1e-5)
    agg = jnp.concatenate([agga_ref[...][0], aggb_ref[...][0]], axis=1)
    hn = _softplus(h_ref[...] + (agg - mu) * inv * g_ref[...] + b_ref[...])
    hn_ref[...] = hn
    if p_ref is not None:
        p_ref[...] = jnp.dot(hn, wp_ref[...], preferred_element_type=_f32)
        q_ref[...] = jnp.dot(hn, wq_ref[...], preferred_element_type=_f32)


def _update_fused(h, agg3, st, g, b, wp, wq):
    bm = 2000
    grid = (N // bm,)
    return pl.pallas_call(
        _update_body,
        grid=grid,
        in_specs=[
            pl.BlockSpec((bm, H), lambda i: (i, 0)),
            pl.BlockSpec((1, bm, HH), lambda i: (0, i, 0)),
            pl.BlockSpec((1, bm, HH), lambda i: (1, i, 0)),
            pl.BlockSpec((8, H), lambda i: (0, 0)),
            pl.BlockSpec((1, H), lambda i: (0, 0)),
            pl.BlockSpec((1, H), lambda i: (0, 0)),
            pl.BlockSpec((H, 2 * H), lambda i: (0, 0)),
            pl.BlockSpec((H, 2 * H), lambda i: (0, 0)),
        ],
        out_specs=[
            pl.BlockSpec((bm, H), lambda i: (i, 0)),
            pl.BlockSpec((bm, 2 * H), lambda i: (i, 0)),
            pl.BlockSpec((bm, 2 * H), lambda i: (i, 0)),
        ],
        out_shape=[
            jax.ShapeDtypeStruct((N, H), _f32),
            jax.ShapeDtypeStruct((N, 2 * H), _f32),
            jax.ShapeDtypeStruct((N, 2 * H), _f32),
        ],
    )(h, agg3, agg3, st, g, b, wp, wq)


def _update_last_body(h_ref, agga_ref, aggb_ref, st_ref, g_ref, b_ref, hn_ref):
    _update_body(h_ref, agga_ref, aggb_ref, st_ref, g_ref, b_ref, None, None,
                 hn_ref, None, None)


def _update_last(h, agg3, st, g, b):
    bm = 2000
    grid = (N // bm,)
    return pl.pallas_call(
        _update_last_body,
        grid=grid,
        in_specs=[
            pl.BlockSpec((bm, H), lambda i: (i, 0)),
            pl.BlockSpec((1, bm, HH), lambda i: (0, i, 0)),
            pl.BlockSpec((1, bm, HH), lambda i: (1, i, 0)),
            pl.BlockSpec((8, H), lambda i: (0, 0)),
            pl.BlockSpec((1, H), lambda i: (0, 0)),
            pl.BlockSpec((1, H), lambda i: (0, 0)),
        ],
        out_specs=pl.BlockSpec((bm, H), lambda i: (i, 0)),
        out_shape=jax.ShapeDtypeStruct((N, H), _f32),
    )(h, agg3, agg3, st, g, b)


def _pool_body(h_ref, gi_ref, pool_ref, cnt_ref):
    @pl.when(pl.program_id(0) == 0)
    def _():
        pool_ref[...] = jnp.zeros((B, H), _f32)
        cnt_ref[...] = jnp.zeros((B, H), _f32)

    gi2 = gi_ref[...][0]
    seg = lax.broadcasted_iota(jnp.int32, (B, 1), 0)
    onehot_t = (seg == gi2).astype(_f32)
    pool_ref[...] += jnp.dot(onehot_t, h_ref[...], preferred_element_type=_f32)
    cnt_col = jnp.sum(onehot_t, axis=1, keepdims=True)
    cnt_ref[...] += jnp.broadcast_to(cnt_col, (B, H))


def _pool(h, gi3):
    bm = 2000
    grid = (N // bm,)
    return pl.pallas_call(
        _pool_body,
        grid=grid,
        in_specs=[
            pl.BlockSpec((bm, H), lambda i: (i, 0)),
            pl.BlockSpec((1, 1, bm), lambda i: (i, 0, 0)),
        ],
        out_specs=[
            pl.BlockSpec((B, H), lambda i: (0, 0)),
            pl.BlockSpec((B, H), lambda i: (0, 0)),
        ],
        out_shape=[
            jax.ShapeDtypeStruct((B, H), _f32),
            jax.ShapeDtypeStruct((B, H), _f32),
        ],
    )(h, gi3)


def _mlp_body(pool_ref, cnt_ref, w1_ref, b1_ref, w2_ref, b2_ref, w3_ref, b3_ref,
              out_ref):
    cnt = jnp.maximum(cnt_ref[...], 1.0)[:, 0:1]
    x = pool_ref[...] / cnt
    x1 = _softplus(jnp.dot(x, w1_ref[...], preferred_element_type=_f32) + b1_ref[...])
    x2 = _softplus(jnp.dot(x1, w2_ref[...], preferred_element_type=_f32) + b2_ref[...])
    out_ref[...] = jnp.dot(x2, w3_ref[...], preferred_element_type=_f32) + b3_ref[...]


def _mlp(pool, cnt, w1, b1, w2, b2, w3p, b3p):
    return pl.pallas_call(
        _mlp_body,
        out_shape=jax.ShapeDtypeStruct((B, 128), _f32),
    )(pool, cnt, w1, b1, w2, b2, w3p, b3p)


def kernel(node_features, edge_index, edge_features, graph_index, embed_W,
           embed_b, gate_W, gate_b, cand_W, cand_b, bn_g, bn_b, mlp1_W,
           mlp1_b, mlp2_W, mlp2_b, mlp3_W, mlp3_b):
    src = edge_index[0]
    dst = edge_index[1]

    wps = [jnp.concatenate([gate_W[l, :H, :], cand_W[l, :H, :]], axis=1)
           for l in range(NC)]
    wqs = [jnp.concatenate([gate_W[l, H:2 * H, :], cand_W[l, H:2 * H, :]], axis=1)
           for l in range(NC)]
    wegs = [gate_W[l, 2 * H:, :] for l in range(NC)]
    wecs = [cand_W[l, 2 * H:, :] for l in range(NC)]

    zrows = jnp.zeros((CS, HH), _f32)
    gi3 = graph_index.reshape(N // 2000, 1, 2000)
    lidx = _lidx(src.reshape(E // 6400, 1, 6400)).reshape(2 * E)

    h, p, q = _embed_fused(node_features, embed_W, embed_b.reshape(1, H),
                           wps[0], wqs[0])

    for l in range(NC):
        s1, s2 = _sc_gather(src, dst, p, q)
        m3 = _edge_elementwise(s1, s2, edge_features, wegs[l], wecs[l],
                               gate_b[l].reshape(1, H),
                               cand_b[l].reshape(1, H))
        agg3 = _sc_scatter(lidx, m3, zrows)
        st = _bn_stats(agg3)
        gl = bn_g[l].reshape(1, H)
        bl = bn_b[l].reshape(1, H)
        if l < NC - 1:
            h, p, q = _update_fused(h, agg3, st, gl, bl,
                                    wps[l + 1], wqs[l + 1])
        else:
            h = _update_last(h, agg3, st, gl, bl)

    pool, cnt = _pool(h, gi3)
    w3p = jnp.pad(mlp3_W, ((0, 0), (0, 127)))
    b3p = jnp.pad(mlp3_b, ((0, 127))).reshape(1, 128)
    y = _mlp(pool, cnt, mlp1_W, mlp1_b.reshape(1, 128), mlp2_W,
             mlp2_b.reshape(1, H), w3p, b3p)
    return y[:, 0]
